# baseline (speedup 1.0000x reference)
import functools

import numpy as np
import jax
import jax.numpy as jnp
from jax import lax
from jax.experimental import pallas as pl
from jax.experimental.pallas import tpu as pltpu
from jax.sharding import Mesh, PartitionSpec as P

_E = 64
_N_ITER = 10
_BM = 512


def _body(x_ref, ew1_ref, eb1_ref, ew2_ref, eb2t_ref, hopbf_ref, wb_ref,
          nw1x_ref, nw1r_ref, nb1_ref, nw2_ref, nb2_ref, nw3_ref, nb3_ref,
          out_ref, state_ref, *, bm):
    f32 = jnp.float32
    bf16 = jnp.bfloat16
    xb = x_ref[...]
    h1 = jax.nn.relu(jnp.dot(xb, ew1_ref[...],
                             preferred_element_type=f32) + eb1_ref[...])
    enc_t = lax.dot_general(ew2_ref[...], h1.astype(bf16),
                            (((0,), (1,)), ((), ())),
                            preferred_element_type=f32)
    enc_t = enc_t + pltpu.repeat(eb2t_ref[...], bm // 128, axis=1)
    state_t = jnp.where(enc_t > 0, 1.0, -1.0)
    state_ref[...] = state_t

    act_t = jnp.dot(hopbf_ref[...], state_t.astype(bf16),
                    preferred_element_type=f32)

    def sweep(_, act):
        for i in range(_E):
            a = act[i:i + 1, :]
            old = state_ref[i:i + 1, :]
            new = jnp.where(a > 0, 1.0, -1.0)
            d = new - old
            state_ref[i:i + 1, :] = new
            wcol = pltpu.repeat(wb_ref[i], bm // 128,
                                axis=1).astype(f32)
            act = act + wcol * d
        return act

    lax.fori_loop(0, _N_ITER, sweep, act_t)
    retr_t = jnp.where(state_ref[...] > 0, 1.0, 0.0)

    h = jax.nn.relu(
        jnp.dot(xb, nw1x_ref[...], preferred_element_type=f32)
        + lax.dot_general(retr_t.astype(bf16), nw1r_ref[...],
                          (((0,), (0,)), ((), ())),
                          preferred_element_type=f32)
        + nb1_ref[...])
    h = jax.nn.relu(jnp.dot(h.astype(bf16), nw2_ref[...],
                            preferred_element_type=f32) + nb2_ref[...])
    out_ref[...] = jnp.dot(h.astype(bf16), nw3_ref[...],
                           preferred_element_type=f32) + nb3_ref[...]


def _pallas(x, e_w1, e_b1, e_w2, eb2t, hop_bf, wb, n_w1x, n_w1r, n_b1, n_w2,
            n_b2, n_w3, n_b3):
    b, in_dim = x.shape
    hid = e_w1.shape[1]
    out_dim = n_w3.shape[1]
    bm = _BM if b % _BM == 0 else b
    nb = b // bm

    const = lambda *bs: pl.BlockSpec(bs, lambda i: tuple(0 for _ in bs))
    return pl.pallas_call(
        functools.partial(_body, bm=bm),
        grid=(nb,),
        in_specs=[
            pl.BlockSpec((bm, in_dim), lambda i: (i, 0)),
            const(in_dim, hid),
            const(1, hid),
            const(hid, _E),
            const(_E, 128),
            const(_E, _E),
            const(_E, _E, 128),
            const(in_dim, hid),
            const(_E, hid),
            const(1, hid),
            const(hid, hid),
            const(1, hid),
            const(hid, out_dim),
            const(1, out_dim),
        ],
        out_specs=pl.BlockSpec((bm, out_dim), lambda i: (i, 0)),
        out_shape=jax.ShapeDtypeStruct((b, out_dim), jnp.float32),
        scratch_shapes=[pltpu.VMEM((_E, bm), jnp.float32)],
        compiler_params=pltpu.CompilerParams(
            dimension_semantics=("arbitrary",),
        ),
        name="hopfield_dqn",
    )(x, e_w1, e_b1, e_w2, eb2t, hop_bf, wb, n_w1x, n_w1r, n_b1, n_w2, n_b2,
      n_w3, n_b3)


def kernel(x, e_w1, e_b1, e_w2, e_b2, hop_w, n_w1, n_b1, n_w2, n_b2, n_w3,
           n_b3):
    b, in_dim = x.shape
    bf16 = jnp.bfloat16

    hop_bf = hop_w.astype(bf16)
    wb = jnp.broadcast_to(hop_w.T[:, :, None], (_E, _E, 128)).astype(bf16)
    eb2t = jnp.broadcast_to(e_b2[:, None], (_E, 128))

    args = (x.astype(bf16), e_w1.astype(bf16), e_b1[None, :],
            e_w2.astype(bf16), eb2t, hop_bf, wb,
            n_w1[:in_dim].astype(bf16), n_w1[in_dim:].astype(bf16),
            n_b1[None, :], n_w2.astype(bf16), n_b2[None, :],
            n_w3.astype(bf16), n_b3[None, :])

    devs = jax.devices()
    n_dev = 2 if (len(devs) >= 2 and b % (2 * _BM) == 0) else 1
    if n_dev == 1:
        return _pallas(*args)

    mesh = Mesh(np.array(devs[:2]), ("b",))
    shard = jax.shard_map(
        _pallas, mesh=mesh,
        in_specs=(P("b"),) + (P(),) * 13,
        out_specs=P("b"),
        check_vma=False,
    )
    return shard(*args)


# speedup vs baseline: 1.1330x; 1.1330x over previous
import functools

import jax
import jax.numpy as jnp
from jax import lax
from jax.experimental import pallas as pl
from jax.experimental.pallas import tpu as pltpu

_E = 64
_N_ITER = 10
_BM = 512


def _body(x_ref, ew1_ref, eb1_ref, ew2_ref, eb2t_ref, hopbf_ref, wb_ref,
          nw1x_ref, nw1r_ref, nb1_ref, nw2_ref, nb2_ref, nw3_ref, nb3_ref,
          out_ref, state_ref, *, bm):
    f32 = jnp.float32
    bf16 = jnp.bfloat16
    xb = x_ref[...]
    h1 = jax.nn.relu(jnp.dot(xb, ew1_ref[...],
                             preferred_element_type=f32) + eb1_ref[...])
    enc_t = lax.dot_general(ew2_ref[...], h1.astype(bf16),
                            (((0,), (1,)), ((), ())),
                            preferred_element_type=f32)
    enc_t = enc_t + pltpu.repeat(eb2t_ref[...], bm // 128, axis=1)
    state_t = jnp.where(enc_t > 0, 1.0, -1.0)
    state_ref[...] = state_t

    act_t = jnp.dot(hopbf_ref[...], state_t.astype(bf16),
                    preferred_element_type=f32)

    def sweep(_, act):
        for i in range(_E):
            a = act[i:i + 1, :]
            old = state_ref[i:i + 1, :]
            new = jnp.where(a > 0, 1.0, -1.0)
            d = new - old
            state_ref[i:i + 1, :] = new
            wcol = pltpu.repeat(wb_ref[i], bm // 128, axis=1)
            act = act + wcol * d
        return act

    lax.fori_loop(0, _N_ITER, sweep, act_t)
    retr_t = jnp.where(state_ref[...] > 0, 1.0, 0.0)

    h = jax.nn.relu(
        jnp.dot(xb, nw1x_ref[...], preferred_element_type=f32)
        + lax.dot_general(retr_t.astype(bf16), nw1r_ref[...],
                          (((0,), (0,)), ((), ())),
                          preferred_element_type=f32)
        + nb1_ref[...])
    h = jax.nn.relu(jnp.dot(h.astype(bf16), nw2_ref[...],
                            preferred_element_type=f32) + nb2_ref[...])
    out_ref[...] = jnp.dot(h.astype(bf16), nw3_ref[...],
                           preferred_element_type=f32) + nb3_ref[...]


def kernel(x, e_w1, e_b1, e_w2, e_b2, hop_w, n_w1, n_b1, n_w2, n_b2, n_w3,
           n_b3, *, interpret=False):
    b, in_dim = x.shape
    hid = e_w1.shape[1]
    out_dim = n_w3.shape[1]
    bm = _BM if b % _BM == 0 else b
    nb = b // bm
    bf16 = jnp.bfloat16

    hop_bf = hop_w.astype(bf16)
    wb_u = lax.bitcast_convert_type(
        jnp.broadcast_to(hop_w.T[:, :, None], (_E, _E, 128)), jnp.uint32)
    wb_u = (wb_u + jnp.uint32(0x7FFF) + ((wb_u >> 16) & jnp.uint32(1)))
    wb = lax.bitcast_convert_type(wb_u & jnp.uint32(0xFFFF0000), jnp.float32)
    eb2t = jnp.broadcast_to(e_b2[:, None], (_E, 128))

    n_w1x = n_w1[:in_dim]
    n_w1r = n_w1[in_dim:]

    const = lambda *bs: pl.BlockSpec(bs, lambda i: tuple(0 for _ in bs))
    return pl.pallas_call(
        functools.partial(_body, bm=bm),
        grid=(nb,),
        in_specs=[
            pl.BlockSpec((bm, in_dim), lambda i: (i, 0)),
            const(in_dim, hid),
            const(1, hid),
            const(hid, _E),
            const(_E, 128),
            const(_E, _E),
            const(_E, _E, 128),
            const(in_dim, hid),
            const(_E, hid),
            const(1, hid),
            const(hid, hid),
            const(1, hid),
            const(hid, out_dim),
            const(1, out_dim),
        ],
        out_specs=pl.BlockSpec((bm, out_dim), lambda i: (i, 0)),
        out_shape=jax.ShapeDtypeStruct((b, out_dim), jnp.float32),
        scratch_shapes=[pltpu.VMEM((_E, bm), jnp.float32)],
        compiler_params=pltpu.CompilerParams(
            dimension_semantics=("arbitrary",),
        ),
        name="hopfield_dqn",
        interpret=interpret,
    )(x.astype(bf16), e_w1.astype(bf16), e_b1[None, :], e_w2.astype(bf16),
      eb2t, hop_bf, wb, n_w1x.astype(bf16), n_w1r.astype(bf16),
      n_b1[None, :], n_w2.astype(bf16), n_b2[None, :], n_w3.astype(bf16),
      n_b3[None, :])
